# Initial kernel scaffold; baseline (speedup 1.0000x reference)
import jax
import jax.numpy as jnp
from jax.experimental import pallas as pl


def kernel(feat, row, col, W, b, a_l, a_r):
    raise NotImplementedError("write your pallas kernel here")


# speedup vs baseline: 21.0674x; 21.0674x over previous
import functools

import jax
import jax.numpy as jnp
from jax import lax
from jax.experimental import pallas as pl
from jax.experimental.pallas import tpu as pltpu
from jax.experimental.pallas import tpu_sc as plsc

N = 10000
E = 320000
D = 128
NEG = 0.2

NC, NS, L = 2, 16, 16
NW = NC * NS
EPW = E // NW
K = 80
NCH = EPW // K
NACC = 10240
NPT = NACC // NS


def _prep_body(feat_ref, wt_ref, b_ref, al_ref, ar_ref,
               z_ref, el_ref, er_ref, c_ref):
    z = jnp.dot(feat_ref[...], wt_ref[...],
                preferred_element_type=jnp.float32) + b_ref[...]
    z_ref[...] = z
    el = jnp.dot(z, al_ref[...], preferred_element_type=jnp.float32)
    er = jnp.dot(z, ar_ref[...], preferred_element_type=jnp.float32)
    el_ref[...] = el
    er_ref[...] = er
    c = jnp.maximum(jnp.max(el) + jnp.max(er), 0.0)
    c_ref[...] = jnp.full((1, 1), 0.0) + c


def _edge_body(row_hbm, col_hbm, elf_hbm, erf_hbm, cvec_hbm, z_hbm,
               zeros_hbm, zeros1_hbm,
               num_hbm, den_hbm,
               row_v, col_v, w_v, el_v, er_v, c_v, rows_v, accum, dacc, sem):
    cid = lax.axis_index("c")
    sid = lax.axis_index("s")
    wid = sid * NC + cid

    pltpu.sync_copy(elf_hbm, el_v)
    pltpu.sync_copy(erf_hbm, er_v)
    pltpu.sync_copy(cvec_hbm, c_v)

    pltpu.sync_copy(zeros_hbm, accum.at[pl.ds(sid * NPT, NPT)])
    pltpu.sync_copy(zeros1_hbm, dacc.at[pl.ds(sid * NPT, NPT)])

    cvec = c_v[...]

    plsc.subcore_barrier()

    def chunk_body(j, carry):
        ch = wid * NCH + j
        pltpu.sync_copy(row_hbm.at[ch], row_v.at[0])
        pltpu.sync_copy(col_hbm.at[ch], col_v.at[0])

        for t in range(K // L):
            ridx = row_v[0, pl.ds(t * L, L)]
            cidx = col_v[0, pl.ds(t * L, L)]
            x = plsc.load_gather(el_v, [ridx]) + plsc.load_gather(er_v, [cidx])
            a = jnp.maximum(x, x * NEG) - cvec
            w_v[0, pl.ds(t * L, L)] = jnp.exp(a)

        pltpu.async_copy(z_hbm.at[col_v.at[0]], rows_v, sem).wait()

        def scale_body(e, c2):
            wsp = plsc.load_gather(
                w_v, [jnp.full((L,), 0, jnp.int32), jnp.full((L,), e, jnp.int32)])
            for q in range(D // L):
                rows_v[e, pl.ds(q * L, L)] = rows_v[e, pl.ds(q * L, L)] * wsp
            return c2

        lax.fori_loop(0, K, scale_body, 0)
        pltpu.sync_copy(rows_v, accum.at[row_v.at[0]], add=True)
        pltpu.sync_copy(w_v.at[0], dacc.at[row_v.at[0]], add=True)
        return carry

    lax.fori_loop(0, NCH, chunk_body, 0)

    plsc.subcore_barrier()
    pltpu.sync_copy(accum.at[pl.ds(sid * NPT, NPT)],
                    num_hbm.at[cid, pl.ds(sid * NPT, NPT)])
    pltpu.sync_copy(dacc.at[pl.ds(sid * NPT, NPT)],
                    den_hbm.at[cid, pl.ds(sid * NPT, NPT)])


_edge_kernel = functools.partial(
    pl.kernel,
    out_type=[
        jax.ShapeDtypeStruct((NC, NACC, D), jnp.float32),
        jax.ShapeDtypeStruct((NC, NACC), jnp.float32),
    ],
    mesh=plsc.VectorSubcoreMesh(core_axis_name="c", subcore_axis_name="s"),
    compiler_params=pltpu.CompilerParams(needs_layout_passes=False),
    scratch_types=[
        pltpu.VMEM((1, K), jnp.int32),
        pltpu.VMEM((1, K), jnp.int32),
        pltpu.VMEM((1, K), jnp.float32),
        pltpu.VMEM((N,), jnp.float32),
        pltpu.VMEM((N,), jnp.float32),
        pltpu.VMEM((L,), jnp.float32),
        pltpu.VMEM((K, D), jnp.float32),
        pltpu.VMEM_SHARED((NACC, D), jnp.float32),
        pltpu.VMEM_SHARED((NACC,), jnp.float32),
        pltpu.SemaphoreType.DMA,
    ],
)(_edge_body)


def _combine_body(p_ref, d_ref, o_ref):
    num = p_ref[0] + p_ref[1]
    den = d_ref[0] + d_ref[1]
    o_ref[...] = jnp.where(den > 0.0, num / den, 0.0)


@jax.jit
def kernel(feat, row, col, W, b, a_l, a_r):
    z, el, er, c = pl.pallas_call(
        _prep_body,
        out_shape=[
            jax.ShapeDtypeStruct((N, D), jnp.float32),
            jax.ShapeDtypeStruct((N, 1), jnp.float32),
            jax.ShapeDtypeStruct((N, 1), jnp.float32),
            jax.ShapeDtypeStruct((1, 1), jnp.float32),
        ],
    )(feat, W.T, b.reshape(1, D), a_l.reshape(D, 1), a_r.reshape(D, 1))

    cvec = jnp.full((L,), 0.0, jnp.float32) + c[0, 0]
    row2 = row.reshape(NW * NCH, K)
    col2 = col.reshape(NW * NCH, K)
    zeros = jnp.zeros((NPT, D), jnp.float32)
    zeros1 = jnp.zeros((NPT,), jnp.float32)

    num, den = _edge_kernel(row2, col2, el.reshape(N), er.reshape(N),
                            cvec, z, zeros, zeros1)

    out = pl.pallas_call(
        _combine_body,
        out_shape=jax.ShapeDtypeStruct((N, D), jnp.float32),
    )(num[:, :N, :], den[:, :N].reshape(NC, N, 1))
    return out


# speedup vs baseline: 26.8908x; 1.2764x over previous
import functools

import jax
import jax.numpy as jnp
from jax import lax
from jax.experimental import pallas as pl
from jax.experimental.pallas import tpu as pltpu
from jax.experimental.pallas import tpu_sc as plsc

N = 10000
E = 320000
D = 128
NEG = 0.2

NC, NS, L = 2, 16, 16
NW = NC * NS
EPW = E // NW
K = 80
NCH = EPW // K
NACC = 10240
NPT = NACC // NS


def _prep_body(feat_ref, wt_ref, b_ref, al_ref, ar_ref,
               z_ref, el_ref, er_ref, c_ref):
    z = jnp.dot(feat_ref[...], wt_ref[...],
                preferred_element_type=jnp.float32) + b_ref[...]
    z_ref[...] = z
    el = jnp.dot(z, al_ref[...], preferred_element_type=jnp.float32)
    er = jnp.dot(z, ar_ref[...], preferred_element_type=jnp.float32)
    el_ref[...] = el
    er_ref[...] = er
    c = jnp.maximum(jnp.max(el) + jnp.max(er), 0.0)
    c_ref[...] = jnp.full((1, 1), 0.0) + c


def _edge_body(row_hbm, col_hbm, elf_hbm, erf_hbm, cvec_hbm, z_hbm,
               zeros_hbm, zeros1_hbm,
               num_hbm, den_hbm,
               row_v, col_v, w_v, el_v, er_v, c_v, rows_v, accum, dacc,
               semi, semz):
    cid = lax.axis_index("c")
    sid = lax.axis_index("s")
    wid = sid * NC + cid

    pltpu.sync_copy(elf_hbm, el_v)
    pltpu.sync_copy(erf_hbm, er_v)
    pltpu.sync_copy(cvec_hbm, c_v)

    pltpu.sync_copy(zeros_hbm, accum.at[pl.ds(sid * NPT, NPT)])
    pltpu.sync_copy(zeros1_hbm, dacc.at[pl.ds(sid * NPT, NPT)])

    cvec = c_v[...]

    def logit(s):
        for t in range(K // L):
            ridx = row_v[s, pl.ds(t * L, L)]
            cidx = col_v[s, pl.ds(t * L, L)]
            x = plsc.load_gather(el_v, [ridx]) + plsc.load_gather(er_v, [cidx])
            a = jnp.maximum(x, x * NEG) - cvec
            w_v[s, pl.ds(t * L, L)] = jnp.exp(a)

    def scale(s):
        def scale_body(e, c2):
            wsp = plsc.load_gather(
                w_v, [jnp.full((L,), s, jnp.int32), jnp.full((L,), e, jnp.int32)])
            for q in range(D // L):
                rows_v[s, e, pl.ds(q * L, L)] = rows_v[s, e, pl.ds(q * L, L)] * wsp
            return c2

        lax.fori_loop(0, K, scale_body, 0)

    def drain_idx(s):
        pltpu.make_async_copy(row_hbm.at[0], row_v.at[s], semi).wait()
        pltpu.make_async_copy(col_hbm.at[0], col_v.at[s], semi).wait()

    def drain_z(s):
        pltpu.make_async_copy(z_hbm.at[pl.ds(0, K)], rows_v.at[s], semz).wait()

    plsc.subcore_barrier()

    base = wid * NCH
    pltpu.sync_copy(row_hbm.at[base], row_v.at[0])
    pltpu.sync_copy(col_hbm.at[base], col_v.at[0])
    logit(0)
    pltpu.async_copy(z_hbm.at[col_v.at[0]], rows_v.at[0], semz)
    pltpu.async_copy(row_hbm.at[base + 1], row_v.at[1], semi)
    pltpu.async_copy(col_hbm.at[base + 1], col_v.at[1], semi)

    def pipe_body(i, carry):
        for b in range(2):
            j = 2 * i + b
            drain_z(b)
            scale(b)
            pltpu.sync_copy(rows_v.at[b], accum.at[row_v.at[b]], add=True)
            pltpu.sync_copy(w_v.at[b], dacc.at[row_v.at[b]], add=True)
            nxt = base + jnp.minimum(j + 2, NCH - 1)
            pltpu.async_copy(row_hbm.at[nxt], row_v.at[b], semi)
            pltpu.async_copy(col_hbm.at[nxt], col_v.at[b], semi)
            drain_idx(1 - b)
            logit(1 - b)
            pltpu.async_copy(z_hbm.at[col_v.at[1 - b]], rows_v.at[1 - b], semz)
        return carry

    lax.fori_loop(0, (NCH - 1) // 2, pipe_body, 0)

    drain_z(0)
    scale(0)
    pltpu.sync_copy(rows_v.at[0], accum.at[row_v.at[0]], add=True)
    pltpu.sync_copy(w_v.at[0], dacc.at[row_v.at[0]], add=True)
    drain_idx(1)

    plsc.subcore_barrier()
    pltpu.sync_copy(accum.at[pl.ds(sid * NPT, NPT)],
                    num_hbm.at[cid, pl.ds(sid * NPT, NPT)])
    pltpu.sync_copy(dacc.at[pl.ds(sid * NPT, NPT)],
                    den_hbm.at[cid, pl.ds(sid * NPT, NPT)])


_edge_kernel = functools.partial(
    pl.kernel,
    out_type=[
        jax.ShapeDtypeStruct((NC, NACC, D), jnp.float32),
        jax.ShapeDtypeStruct((NC, NACC), jnp.float32),
    ],
    mesh=plsc.VectorSubcoreMesh(core_axis_name="c", subcore_axis_name="s"),
    compiler_params=pltpu.CompilerParams(needs_layout_passes=False),
    scratch_types=[
        pltpu.VMEM((2, K), jnp.int32),
        pltpu.VMEM((2, K), jnp.int32),
        pltpu.VMEM((2, K), jnp.float32),
        pltpu.VMEM((N,), jnp.float32),
        pltpu.VMEM((N,), jnp.float32),
        pltpu.VMEM((L,), jnp.float32),
        pltpu.VMEM((2, K, D), jnp.float32),
        pltpu.VMEM_SHARED((NACC, D), jnp.float32),
        pltpu.VMEM_SHARED((NACC,), jnp.float32),
        pltpu.SemaphoreType.DMA,
        pltpu.SemaphoreType.DMA,
    ],
)(_edge_body)


def _combine_body(p_ref, d_ref, o_ref):
    num = p_ref[0] + p_ref[1]
    den = d_ref[0] + d_ref[1]
    o_ref[...] = jnp.where(den > 0.0, num / den, 0.0)


@jax.jit
def kernel(feat, row, col, W, b, a_l, a_r):
    z, el, er, c = pl.pallas_call(
        _prep_body,
        out_shape=[
            jax.ShapeDtypeStruct((N, D), jnp.float32),
            jax.ShapeDtypeStruct((N, 1), jnp.float32),
            jax.ShapeDtypeStruct((N, 1), jnp.float32),
            jax.ShapeDtypeStruct((1, 1), jnp.float32),
        ],
    )(feat, W.T, b.reshape(1, D), a_l.reshape(D, 1), a_r.reshape(D, 1))

    cvec = jnp.full((L,), 0.0, jnp.float32) + c[0, 0]
    row2 = row.reshape(NW * NCH, K)
    col2 = col.reshape(NW * NCH, K)
    zeros = jnp.zeros((NPT, D), jnp.float32)
    zeros1 = jnp.zeros((NPT,), jnp.float32)

    num, den = _edge_kernel(row2, col2, el.reshape(N), er.reshape(N),
                            cvec, z, zeros, zeros1)

    out = pl.pallas_call(
        _combine_body,
        out_shape=jax.ShapeDtypeStruct((N, D), jnp.float32),
    )(num[:, :N, :], den[:, :N].reshape(NC, N, 1))
    return out


# speedup vs baseline: 32.6638x; 1.2147x over previous
import functools

import jax
import jax.numpy as jnp
from jax import lax
from jax.experimental import pallas as pl
from jax.experimental.pallas import tpu as pltpu
from jax.experimental.pallas import tpu_sc as plsc

N = 10000
E = 320000
D = 128
NEG = 0.2

NC, NS, L = 2, 16, 16
NW = NC * NS
EPW = E // NW
K = 80
NCH = EPW // K
NACC = 10240
NPT = NACC // NS


def _prep_body(feat_ref, wt_ref, b_ref, al_ref, ar_ref,
               z_ref, el_ref, er_ref, c_ref):
    z = jnp.dot(feat_ref[...], wt_ref[...],
                preferred_element_type=jnp.float32) + b_ref[...]
    z_ref[...] = z
    el = jnp.dot(z, al_ref[...], preferred_element_type=jnp.float32)
    er = jnp.dot(z, ar_ref[...], preferred_element_type=jnp.float32)
    el_ref[...] = el
    er_ref[...] = er
    c = jnp.maximum(jnp.max(el) + jnp.max(er), 0.0)
    c_ref[...] = jnp.full((1, 1), 0.0) + c


def _edge_body(row_hbm, col_hbm, elf_hbm, erf_hbm, cvec_hbm, z_hbm,
               zeros_hbm, zeros1_hbm,
               num_hbm, den_hbm,
               row_v, col_v, rowsc_v, w_v, el_v, er_v, c_v, rows_v,
               accum, dacc, semi, semz, sems):
    cid = lax.axis_index("c")
    sid = lax.axis_index("s")
    wid = sid * NC + cid

    pltpu.sync_copy(elf_hbm, el_v)
    pltpu.sync_copy(erf_hbm, er_v)
    pltpu.sync_copy(cvec_hbm, c_v)

    pltpu.sync_copy(zeros_hbm, accum.at[pl.ds(sid * NPT, NPT)])
    pltpu.sync_copy(zeros1_hbm, dacc.at[pl.ds(sid * NPT, NPT)])

    cvec = c_v[...]

    def logit(s):
        for t in range(K // L):
            ridx = row_v[s, pl.ds(t * L, L)]
            cidx = col_v[s, pl.ds(t * L, L)]
            x = plsc.load_gather(el_v, [ridx]) + plsc.load_gather(er_v, [cidx])
            a = jnp.maximum(x, x * NEG) - cvec
            w_v[s, pl.ds(t * L, L)] = jnp.exp(a)

    def scale(s):
        for t in range(K // L):
            rowsc_v[s, pl.ds(t * L, L)] = row_v[s, pl.ds(t * L, L)]

        def scale_body(e, c2):
            wsp = plsc.load_gather(
                w_v, [jnp.full((L,), s, jnp.int32), jnp.full((L,), e, jnp.int32)])
            for q in range(D // L):
                rows_v[s, e, pl.ds(q * L, L)] = rows_v[s, e, pl.ds(q * L, L)] * wsp
            return c2

        lax.fori_loop(0, K, scale_body, 0)

    def scatter(s):
        pltpu.async_copy(rows_v.at[s], accum.at[rowsc_v.at[s]], sems, add=True)
        pltpu.async_copy(w_v.at[s], dacc.at[rowsc_v.at[s]], sems, add=True)

    def drain_scatter(s):
        pltpu.make_async_copy(z_hbm.at[pl.ds(0, K)], rows_v.at[s], sems).wait()
        pltpu.make_async_copy(elf_hbm.at[pl.ds(0, K)], w_v.at[s], sems).wait()

    def drain_idx(s):
        pltpu.make_async_copy(row_hbm.at[0], row_v.at[s], semi).wait()
        pltpu.make_async_copy(col_hbm.at[0], col_v.at[s], semi).wait()

    def drain_z(s):
        pltpu.make_async_copy(z_hbm.at[pl.ds(0, K)], rows_v.at[s], semz).wait()

    plsc.subcore_barrier()

    base = wid * NCH
    pltpu.sync_copy(row_hbm.at[base], row_v.at[0])
    pltpu.sync_copy(col_hbm.at[base], col_v.at[0])
    logit(0)
    pltpu.async_copy(z_hbm.at[col_v.at[0]], rows_v.at[0], semz)
    pltpu.async_copy(row_hbm.at[base + 1], row_v.at[1], semi)
    pltpu.async_copy(col_hbm.at[base + 1], col_v.at[1], semi)

    def pipe_body(i, carry):
        for b in range(2):
            j = 2 * i + b
            drain_z(b)
            scale(b)

            @pl.when(j > 0)
            def _():
                drain_scatter(1 - b)

            scatter(b)
            drain_idx(1 - b)
            pltpu.async_copy(z_hbm.at[col_v.at[1 - b]], rows_v.at[1 - b], semz)
            nxt = base + jnp.minimum(j + 2, NCH - 1)
            pltpu.async_copy(row_hbm.at[nxt], row_v.at[b], semi)
            pltpu.async_copy(col_hbm.at[nxt], col_v.at[b], semi)
            logit(1 - b)
        return carry

    lax.fori_loop(0, (NCH - 1) // 2, pipe_body, 0)

    drain_z(0)
    scale(0)
    drain_scatter(1)
    scatter(0)
    drain_idx(1)
    drain_scatter(0)

    plsc.subcore_barrier()
    pltpu.sync_copy(accum.at[pl.ds(sid * NPT, NPT)],
                    num_hbm.at[cid, pl.ds(sid * NPT, NPT)])
    pltpu.sync_copy(dacc.at[pl.ds(sid * NPT, NPT)],
                    den_hbm.at[cid, pl.ds(sid * NPT, NPT)])


_edge_kernel = functools.partial(
    pl.kernel,
    out_type=[
        jax.ShapeDtypeStruct((NC, NACC, D), jnp.float32),
        jax.ShapeDtypeStruct((NC, NACC), jnp.float32),
    ],
    mesh=plsc.VectorSubcoreMesh(core_axis_name="c", subcore_axis_name="s"),
    compiler_params=pltpu.CompilerParams(needs_layout_passes=False),
    scratch_types=[
        pltpu.VMEM((2, K), jnp.int32),
        pltpu.VMEM((2, K), jnp.int32),
        pltpu.VMEM((2, K), jnp.int32),
        pltpu.VMEM((2, K), jnp.float32),
        pltpu.VMEM((N,), jnp.float32),
        pltpu.VMEM((N,), jnp.float32),
        pltpu.VMEM((L,), jnp.float32),
        pltpu.VMEM((2, K, D), jnp.float32),
        pltpu.VMEM_SHARED((NACC, D), jnp.float32),
        pltpu.VMEM_SHARED((NACC,), jnp.float32),
        pltpu.SemaphoreType.DMA,
        pltpu.SemaphoreType.DMA,
        pltpu.SemaphoreType.DMA,
    ],
)(_edge_body)


def _combine_body(p_ref, d_ref, o_ref):
    num = p_ref[0] + p_ref[1]
    den = d_ref[0] + d_ref[1]
    o_ref[...] = jnp.where(den > 0.0, num / den, 0.0)


@jax.jit
def kernel(feat, row, col, W, b, a_l, a_r):
    z, el, er, c = pl.pallas_call(
        _prep_body,
        out_shape=[
            jax.ShapeDtypeStruct((N, D), jnp.float32),
            jax.ShapeDtypeStruct((N, 1), jnp.float32),
            jax.ShapeDtypeStruct((N, 1), jnp.float32),
            jax.ShapeDtypeStruct((1, 1), jnp.float32),
        ],
    )(feat, W.T, b.reshape(1, D), a_l.reshape(D, 1), a_r.reshape(D, 1))

    cvec = jnp.full((L,), 0.0, jnp.float32) + c[0, 0]
    row2 = row.reshape(NW * NCH, K)
    col2 = col.reshape(NW * NCH, K)
    zeros = jnp.zeros((NPT, D), jnp.float32)
    zeros1 = jnp.zeros((NPT,), jnp.float32)

    num, den = _edge_kernel(row2, col2, el.reshape(N), er.reshape(N),
                            cvec, z, zeros, zeros1)

    out = pl.pallas_call(
        _combine_body,
        out_shape=jax.ShapeDtypeStruct((N, D), jnp.float32),
    )(num[:, :N, :], den[:, :N].reshape(NC, N, 1))
    return out


# speedup vs baseline: 35.0622x; 1.0734x over previous
import functools

import jax
import jax.numpy as jnp
from jax import lax
from jax.experimental import pallas as pl
from jax.experimental.pallas import tpu as pltpu
from jax.experimental.pallas import tpu_sc as plsc

N = 10000
E = 320000
D = 128
NEG = 0.2

NC, NS, L = 2, 16, 16
NW = NC * NS
EPW = E // NW
K = 80
NCH = EPW // K
NACC = 10240
NPT = NACC // NS
NLAST = N - (NS - 1) * NPT
NDEN = 10112
DLAST = NDEN - (NS - 1) * NPT


def _prep_body(feat_ref, wt_ref, b_ref, al_ref, ar_ref,
               z_ref, el_ref, er_ref, c_ref):
    z = jnp.dot(feat_ref[...], wt_ref[...],
                preferred_element_type=jnp.float32) + b_ref[...]
    z_ref[...] = z
    el = jnp.dot(z, al_ref[...], preferred_element_type=jnp.float32)
    er = jnp.dot(z, ar_ref[...], preferred_element_type=jnp.float32)
    el_ref[...] = el
    er_ref[...] = er
    c = jnp.maximum(jnp.max(el) + jnp.max(er), 0.0)
    c_ref[...] = jnp.full((1, L), 0.0) + c


def _edge_body(row_hbm, col_hbm, elf_hbm, erf_hbm, cvec_hbm, z_hbm,
               num_hbm, den_hbm,
               row_v, col_v, rowsc_v, w_v, el_v, er_v, c_v, rows_v,
               accum, dacc, semi, semz, sems):
    cid = lax.axis_index("c")
    sid = lax.axis_index("s")
    wid = sid * NC + cid

    pltpu.sync_copy(elf_hbm, el_v)
    pltpu.sync_copy(erf_hbm, er_v)
    pltpu.sync_copy(cvec_hbm, c_v)

    cvec = c_v[...]

    zv = cvec * 0.0

    def zrow_body(e, carry):
        for q in range(D // L):
            rows_v[1, e, pl.ds(q * L, L)] = zv
        return carry

    lax.fori_loop(0, K, zrow_body, 0)
    for t in range(K // L):
        w_v[0, pl.ds(t * L, L)] = zv
    for q in range(NPT // K):
        pltpu.sync_copy(rows_v.at[1], accum.at[pl.ds(sid * NPT + q * K, K)])
        pltpu.sync_copy(w_v.at[0], dacc.at[pl.ds(sid * NPT + q * K, K)])

    def logit(s):
        for t in range(K // L):
            ridx = row_v[s, pl.ds(t * L, L)]
            cidx = col_v[s, pl.ds(t * L, L)]
            x = plsc.load_gather(el_v, [ridx]) + plsc.load_gather(er_v, [cidx])
            a = jnp.maximum(x, x * NEG) - cvec
            w_v[s, pl.ds(t * L, L)] = jnp.exp(a)

    def scale(s):
        for t in range(K // L):
            rowsc_v[s, pl.ds(t * L, L)] = row_v[s, pl.ds(t * L, L)]

        def scale_body(i, c2):
            for u in range(4):
                e = i * 4 + u
                wsp = plsc.load_gather(
                    w_v,
                    [jnp.full((L,), s, jnp.int32), jnp.full((L,), e, jnp.int32)])
                for q in range(D // L):
                    rows_v[s, e, pl.ds(q * L, L)] = (
                        rows_v[s, e, pl.ds(q * L, L)] * wsp)
            return c2

        lax.fori_loop(0, K // 4, scale_body, 0)

    def scatter(s):
        pltpu.async_copy(rows_v.at[s], accum.at[rowsc_v.at[s]], sems, add=True)
        pltpu.async_copy(w_v.at[s], dacc.at[rowsc_v.at[s]], sems, add=True)

    def drain_scatter(s):
        pltpu.make_async_copy(z_hbm.at[pl.ds(0, K)], rows_v.at[s], sems).wait()
        pltpu.make_async_copy(elf_hbm.at[pl.ds(0, K)], w_v.at[s], sems).wait()

    def drain_idx(s):
        pltpu.make_async_copy(row_hbm.at[0], row_v.at[s], semi).wait()
        pltpu.make_async_copy(col_hbm.at[0], col_v.at[s], semi).wait()

    def drain_z(s):
        pltpu.make_async_copy(z_hbm.at[pl.ds(0, K)], rows_v.at[s], semz).wait()

    plsc.subcore_barrier()

    base = wid * NCH
    pltpu.sync_copy(row_hbm.at[base], row_v.at[0])
    pltpu.sync_copy(col_hbm.at[base], col_v.at[0])
    logit(0)
    pltpu.async_copy(z_hbm.at[col_v.at[0]], rows_v.at[0], semz)
    pltpu.async_copy(row_hbm.at[base + 1], row_v.at[1], semi)
    pltpu.async_copy(col_hbm.at[base + 1], col_v.at[1], semi)

    def pipe_body(i, carry):
        for b in range(2):
            j = 2 * i + b
            drain_z(b)
            scale(b)

            @pl.when(j > 0)
            def _():
                drain_scatter(1 - b)

            scatter(b)
            drain_idx(1 - b)
            pltpu.async_copy(z_hbm.at[col_v.at[1 - b]], rows_v.at[1 - b], semz)
            nxt = base + jnp.minimum(j + 2, NCH - 1)
            pltpu.async_copy(row_hbm.at[nxt], row_v.at[b], semi)
            pltpu.async_copy(col_hbm.at[nxt], col_v.at[b], semi)
            logit(1 - b)
        return carry

    lax.fori_loop(0, (NCH - 1) // 2, pipe_body, 0)

    drain_z(0)
    scale(0)
    drain_scatter(1)
    scatter(0)
    drain_idx(1)
    drain_scatter(0)

    plsc.subcore_barrier()

    @pl.when(sid < NS - 1)
    def _():
        pltpu.sync_copy(accum.at[pl.ds(sid * NPT, NPT)],
                        num_hbm.at[cid, pl.ds(sid * NPT, NPT)])
        pltpu.sync_copy(dacc.at[pl.ds(sid * NPT, NPT)],
                        den_hbm.at[cid, pl.ds(sid * NPT, NPT)])

    @pl.when(sid == NS - 1)
    def _():
        pltpu.sync_copy(accum.at[pl.ds((NS - 1) * NPT, NLAST)],
                        num_hbm.at[cid, pl.ds((NS - 1) * NPT, NLAST)])
        pltpu.sync_copy(dacc.at[pl.ds((NS - 1) * NPT, DLAST)],
                        den_hbm.at[cid, pl.ds((NS - 1) * NPT, DLAST)])


_edge_kernel = functools.partial(
    pl.kernel,
    out_type=[
        jax.ShapeDtypeStruct((NC, N, D), jnp.float32),
        jax.ShapeDtypeStruct((NC, NDEN), jnp.float32),
    ],
    mesh=plsc.VectorSubcoreMesh(core_axis_name="c", subcore_axis_name="s"),
    compiler_params=pltpu.CompilerParams(needs_layout_passes=False),
    scratch_types=[
        pltpu.VMEM((2, K), jnp.int32),
        pltpu.VMEM((2, K), jnp.int32),
        pltpu.VMEM((2, K), jnp.int32),
        pltpu.VMEM((2, K), jnp.float32),
        pltpu.VMEM((N,), jnp.float32),
        pltpu.VMEM((N,), jnp.float32),
        pltpu.VMEM((L,), jnp.float32),
        pltpu.VMEM((2, K, D), jnp.float32),
        pltpu.VMEM_SHARED((NACC, D), jnp.float32),
        pltpu.VMEM_SHARED((NACC,), jnp.float32),
        pltpu.SemaphoreType.DMA,
        pltpu.SemaphoreType.DMA,
        pltpu.SemaphoreType.DMA,
    ],
)(_edge_body)


def _combine_body(p_ref, d_ref, o_ref):
    num = p_ref[0] + p_ref[1]
    den = d_ref[0] + d_ref[1]
    o_ref[...] = jnp.where(den > 0.0, num / den, 0.0)


@jax.jit
def kernel(feat, row, col, W, b, a_l, a_r):
    z, el, er, c = pl.pallas_call(
        _prep_body,
        out_shape=[
            jax.ShapeDtypeStruct((N, D), jnp.float32),
            jax.ShapeDtypeStruct((N, 1), jnp.float32),
            jax.ShapeDtypeStruct((N, 1), jnp.float32),
            jax.ShapeDtypeStruct((1, L), jnp.float32),
        ],
    )(feat, W.T, b.reshape(1, D), a_l.reshape(D, 1), a_r.reshape(D, 1))

    row2 = row.reshape(NW * NCH, K)
    col2 = col.reshape(NW * NCH, K)

    num, den = _edge_kernel(row2, col2, el.reshape(N), er.reshape(N),
                            c.reshape(L), z)

    out = pl.pallas_call(
        _combine_body,
        out_shape=jax.ShapeDtypeStruct((N, D), jnp.float32),
    )(num, den[:, :N].reshape(NC, N, 1))
    return out


# speedup vs baseline: 45.3350x; 1.2930x over previous
import functools

import jax
import jax.numpy as jnp
from jax import lax
from jax.experimental import pallas as pl
from jax.experimental.pallas import tpu as pltpu
from jax.experimental.pallas import tpu_sc as plsc

N = 10000
E = 320000
D = 128
NEG = 0.2

NC, NS, L = 2, 16, 16
NW = NC * NS
EPW = E // NW
K = 80
NCH = EPW // K
NACC = 10240
NPT = NACC // NS
NLAST = N - (NS - 1) * NPT
NDEN = 10112
DLAST = NDEN - (NS - 1) * NPT


def _prep_body(feat_ref, wt_ref, b_ref, al_ref, ar_ref,
               z_ref, el_ref, er_ref, c_ref):
    z = jnp.dot(feat_ref[...], wt_ref[...],
                preferred_element_type=jnp.float32) + b_ref[...]
    z_ref[...] = z
    el = jnp.dot(z, al_ref[...], preferred_element_type=jnp.float32)
    er = jnp.dot(z, ar_ref[...], preferred_element_type=jnp.float32)
    el_ref[...] = el
    er_ref[...] = er
    c = jnp.maximum(jnp.max(el) + jnp.max(er), 0.0)
    c_ref[...] = jnp.full((1, L), 0.0) + c


def _edge_body(row_hbm, col_hbm, elf_hbm, erf_hbm, cvec_hbm, z_hbm,
               num_hbm, den_hbm,
               row_v, col_v, rowsc_v, w_v, el_v, er_v, c_v, rows_v,
               accum, dacc, semi, semz, sems):
    cid = lax.axis_index("c")
    sid = lax.axis_index("s")
    wid = sid * NC + cid

    pltpu.sync_copy(elf_hbm, el_v)
    pltpu.sync_copy(erf_hbm, er_v)
    pltpu.sync_copy(cvec_hbm, c_v)

    cvec = c_v[...]

    zv = cvec * 0.0

    def zrow_body(e, carry):
        for q in range(D // L):
            rows_v[1, e, pl.ds(q * L, L)] = zv
        return carry

    lax.fori_loop(0, K, zrow_body, 0)
    for t in range(K // L):
        w_v[0, pl.ds(t * L, L)] = zv
    for q in range(NPT // K):
        pltpu.sync_copy(rows_v.at[1], accum.at[pl.ds(sid * NPT + q * K, K)])
        pltpu.sync_copy(w_v.at[0], dacc.at[pl.ds(sid * NPT + q * K, K)])

    def logit(s):
        for t in range(K // L):
            ridx = row_v[s, pl.ds(t * L, L)]
            cidx = col_v[s, pl.ds(t * L, L)]
            x = plsc.load_gather(el_v, [ridx]) + plsc.load_gather(er_v, [cidx])
            a = jnp.maximum(x, x * NEG) - cvec
            w_v[s, pl.ds(t * L, L)] = jnp.exp(a)

    def scale(s):
        for t in range(K // L):
            rowsc_v[s, pl.ds(t * L, L)] = row_v[s, pl.ds(t * L, L)]

        def scale_body(i, c2):
            for u in range(4):
                e = i * 4 + u
                wsp = plsc.load_gather(
                    w_v,
                    [jnp.full((L,), s, jnp.int32), jnp.full((L,), e, jnp.int32)])
                for q in range(D // L):
                    rows_v[s, e, pl.ds(q * L, L)] = (
                        rows_v[s, e, pl.ds(q * L, L)] * wsp)
            return c2

        lax.fori_loop(0, K // 4, scale_body, 0)

    def scatter(s):
        pltpu.async_copy(rows_v.at[s], accum.at[rowsc_v.at[s]], sems, add=True)
        pltpu.async_copy(w_v.at[s], dacc.at[rowsc_v.at[s]], sems, add=True)

    def drain_scatter(s):
        pltpu.make_async_copy(z_hbm.at[pl.ds(0, K)], rows_v.at[s], sems).wait()
        pltpu.make_async_copy(elf_hbm.at[pl.ds(0, K)], w_v.at[s], sems).wait()

    def drain_idx(s):
        pltpu.make_async_copy(row_hbm.at[0], row_v.at[s], semi).wait()
        pltpu.make_async_copy(col_hbm.at[0], col_v.at[s], semi).wait()

    def drain_z(s):
        pltpu.make_async_copy(z_hbm.at[pl.ds(0, K)], rows_v.at[s], semz).wait()

    plsc.subcore_barrier()

    base = wid * NCH
    pltpu.sync_copy(row_hbm.at[base], row_v.at[0])
    pltpu.sync_copy(col_hbm.at[base], col_v.at[0])
    logit(0)
    pltpu.async_copy(z_hbm.at[col_v.at[0]], rows_v.at[0], semz)
    pltpu.async_copy(row_hbm.at[base + 1], row_v.at[1], semi)
    pltpu.async_copy(col_hbm.at[base + 1], col_v.at[1], semi)

    def pipe_body(i, carry):
        for b in range(2):
            j = 2 * i + b

            @pl.when(j > 0)
            def _():
                drain_scatter(1 - b)

            drain_idx(1 - b)
            pltpu.async_copy(z_hbm.at[col_v.at[1 - b]], rows_v.at[1 - b], semz)
            drain_z(b)
            scale(b)
            scatter(b)
            nxt = base + jnp.minimum(j + 2, NCH - 1)
            pltpu.async_copy(row_hbm.at[nxt], row_v.at[b], semi)
            pltpu.async_copy(col_hbm.at[nxt], col_v.at[b], semi)
            logit(1 - b)
        return carry

    lax.fori_loop(0, (NCH - 1) // 2, pipe_body, 0)

    drain_z(0)
    scale(0)
    drain_scatter(1)
    scatter(0)
    drain_idx(1)
    drain_scatter(0)

    plsc.subcore_barrier()

    @pl.when(sid < NS - 1)
    def _():
        pltpu.sync_copy(accum.at[pl.ds(sid * NPT, NPT)],
                        num_hbm.at[cid, pl.ds(sid * NPT, NPT)])
        pltpu.sync_copy(dacc.at[pl.ds(sid * NPT, NPT)],
                        den_hbm.at[cid, pl.ds(sid * NPT, NPT)])

    @pl.when(sid == NS - 1)
    def _():
        pltpu.sync_copy(accum.at[pl.ds((NS - 1) * NPT, NLAST)],
                        num_hbm.at[cid, pl.ds((NS - 1) * NPT, NLAST)])
        pltpu.sync_copy(dacc.at[pl.ds((NS - 1) * NPT, DLAST)],
                        den_hbm.at[cid, pl.ds((NS - 1) * NPT, DLAST)])


_edge_kernel = functools.partial(
    pl.kernel,
    out_type=[
        jax.ShapeDtypeStruct((NC, N, D), jnp.float32),
        jax.ShapeDtypeStruct((NC, NDEN), jnp.float32),
    ],
    mesh=plsc.VectorSubcoreMesh(core_axis_name="c", subcore_axis_name="s"),
    compiler_params=pltpu.CompilerParams(needs_layout_passes=False),
    scratch_types=[
        pltpu.VMEM((2, K), jnp.int32),
        pltpu.VMEM((2, K), jnp.int32),
        pltpu.VMEM((2, K), jnp.int32),
        pltpu.VMEM((2, K), jnp.float32),
        pltpu.VMEM((N,), jnp.float32),
        pltpu.VMEM((N,), jnp.float32),
        pltpu.VMEM((L,), jnp.float32),
        pltpu.VMEM((2, K, D), jnp.float32),
        pltpu.VMEM_SHARED((NACC, D), jnp.float32),
        pltpu.VMEM_SHARED((NACC,), jnp.float32),
        pltpu.SemaphoreType.DMA,
        pltpu.SemaphoreType.DMA,
        pltpu.SemaphoreType.DMA,
    ],
)(_edge_body)


def _combine_body(p_ref, d_ref, o_ref):
    num = p_ref[0] + p_ref[1]
    den = d_ref[0] + d_ref[1]
    o_ref[...] = jnp.where(den > 0.0, num / den, 0.0)


@jax.jit
def kernel(feat, row, col, W, b, a_l, a_r):
    z, el, er, c = pl.pallas_call(
        _prep_body,
        out_shape=[
            jax.ShapeDtypeStruct((N, D), jnp.float32),
            jax.ShapeDtypeStruct((N, 1), jnp.float32),
            jax.ShapeDtypeStruct((N, 1), jnp.float32),
            jax.ShapeDtypeStruct((1, L), jnp.float32),
        ],
    )(feat, W.T, b.reshape(1, D), a_l.reshape(D, 1), a_r.reshape(D, 1))

    row2 = row.reshape(NW * NCH, K)
    col2 = col.reshape(NW * NCH, K)

    num, den = _edge_kernel(row2, col2, el.reshape(N), er.reshape(N),
                            c.reshape(L), z)

    out = pl.pallas_call(
        _combine_body,
        out_shape=jax.ShapeDtypeStruct((N, D), jnp.float32),
    )(num, den[:, :N].reshape(NC, N, 1))
    return out
